# baseline (speedup 1.0000x reference)
import functools

import jax
import jax.numpy as jnp
from jax import lax
from jax.experimental import pallas as pl
from jax.experimental.pallas import tpu as pltpu
from jax.experimental.pallas import tpu_sc as plsc

NW = 32
SB = 512
TB = 128
NH = SB // TB
NS = 4


def _build(S, D, Bc):
    Sp = S + (-S) % 8
    NT = S * NH
    mesh = plsc.VectorSubcoreMesh(core_axis_name="c", subcore_axis_name="s")

    @functools.partial(
        pl.kernel,
        mesh=mesh,
        out_type=jax.ShapeDtypeStruct((S, D, Bc), jnp.float32),
        compiler_params=pltpu.CompilerParams(
            use_tc_tiling_on_sc=True, needs_layout_passes=False
        ),
        scratch_types=[
            pltpu.VMEM((Sp, SB), jnp.int32),
            pltpu.VMEM((NS, TB), jnp.int32),
            pltpu.VMEM((NS * TB,), jnp.int32),
            pltpu.VMEM((NS * TB, 128), jnp.float32),
            pltpu.VMEM((NS * D, TB), jnp.float32),
        ] + [pltpu.SemaphoreType.DMA] * (2 * NS),
    )
    def body(idx_hbm, w4_hbm, out_hbm, idx_v, idx4_v, off_v, gbuf, tbuf,
             *sems):
        gsem = sems[:NS]
        wsem = sems[NS:]
        wid = lax.axis_index("s") * 2 + lax.axis_index("c")
        b0w = wid * SB

        pltpu.sync_copy(idx_hbm.at[:, pl.ds(b0w, SB)], idx_v)

        iota = lax.iota(jnp.int32, 16)

        def prep_fire(c, slot):
            s_ = c // NH
            boff = (c % NH) * TB
            for g in range(TB // 16):
                v = idx_v[s_, pl.ds(boff + g * 16, 16)]
                idx4_v[slot, pl.ds(g * 16, 16)] = v >> 2
                off_v[pl.ds(slot * TB + g * 16, 16)] = (v & 3) << 5
            pltpu.async_copy(
                w4_hbm.at[idx4_v.at[slot]],
                gbuf.at[pl.ds(slot * TB, TB)],
                gsem[slot],
            )

        def drain_gather(slot):
            pltpu.make_async_copy(
                w4_hbm.at[pl.ds(0, TB)],
                gbuf.at[pl.ds(slot * TB, TB)],
                gsem[slot],
            ).wait()

        def transpose(slot):
            gv = gbuf.at[pl.ds(slot * TB, TB)]
            vjs = [g * 16 + iota for g in range(TB // 16)]
            vcols = [off_v[pl.ds(slot * TB + g * 16, 16)]
                     for g in range(TB // 16)]

            for d in range(D):
                for g in range(TB // 16):
                    v = plsc.load_gather(gv, [vjs[g], vcols[g] + d])
                    tbuf[slot * D + d, pl.ds(g * 16, 16)] = v

        def writeback(c, slot):
            s_ = c // NH
            boff = (c % NH) * TB
            pltpu.async_copy(
                tbuf.at[pl.ds(slot * D, D)],
                out_hbm.at[s_, :, pl.ds(b0w + boff, TB)],
                wsem[slot],
            )

        def drain_wb(slot):
            pltpu.make_async_copy(
                tbuf.at[pl.ds(slot * D, D)],
                out_hbm.at[0, :, pl.ds(0, TB)],
                wsem[slot],
            ).wait()

        for p in range(NS - 1):
            prep_fire(p, p)

        def outer(c, carry):
            for slot in range(NS):
                cc = c + slot
                nxt = cc + NS - 1

                @pl.when(nxt < NT)
                def _():
                    prep_fire(nxt, (slot - 1) % NS)

                drain_gather(slot)

                @pl.when(cc >= NS)
                def _():
                    drain_wb(slot)

                transpose(slot)
                writeback(cc, slot)
            return carry

        lax.fori_loop(0, NT // NS, lambda i, cr: outer(i * NS, cr), 0)
        for p in range(NS):
            drain_wb(p)

    return body


def kernel(token_ids, weights):
    B0, S = token_ids.shape
    V, D = weights.shape
    idxp = jnp.pad(token_ids.T.astype(jnp.int32), ((0, (-S) % 8), (0, 0)))
    w4 = weights.reshape(V * D // 128, 128)
    oT = _build(S, D, B0)(idxp, w4)
    return oT.transpose(2, 0, 1)


# speedup vs baseline: 1.2004x; 1.2004x over previous
import functools

import jax
import jax.numpy as jnp
from jax import lax
from jax.experimental import pallas as pl
from jax.experimental.pallas import tpu as pltpu
from jax.experimental.pallas import tpu_sc as plsc

NW = 32
SB = 512
TB = 128
NH = SB // TB
NS = 4


def _build(S, D, Bc):
    Sp = S + (-S) % 8
    NT = S * NH
    mesh = plsc.VectorSubcoreMesh(core_axis_name="c", subcore_axis_name="s")

    @functools.partial(
        pl.kernel,
        mesh=mesh,
        out_type=jax.ShapeDtypeStruct((S, D, Bc), jnp.float32),
        compiler_params=pltpu.CompilerParams(
            use_tc_tiling_on_sc=True, needs_layout_passes=False
        ),
        scratch_types=[
            pltpu.VMEM((Sp, SB), jnp.int32),
            pltpu.VMEM((NS, TB), jnp.int32),
            pltpu.VMEM((NS * TB,), jnp.int32),
            pltpu.VMEM((NS * TB, 128), jnp.float32),
            pltpu.VMEM((NS * D, TB), jnp.float32),
        ] + [pltpu.SemaphoreType.DMA] * (2 * NS),
    )
    def body(idx_hbm, w4_hbm, out_hbm, idx_v, idx4_v, off_v, gbuf, tbuf,
             *sems):
        gsem = sems[:NS]
        wsem = sems[NS:]
        wid = lax.axis_index("s") * 2 + lax.axis_index("c")
        b0w = wid * SB

        pltpu.sync_copy(idx_hbm.at[:, pl.ds(b0w, SB)], idx_v)

        iota = lax.iota(jnp.int32, 16)

        def prep_fire(c, slot):
            s_ = c // NH
            boff = (c % NH) * TB
            for g in range(TB // 16):
                v = idx_v[s_, pl.ds(boff + g * 16, 16)]
                idx4_v[slot, pl.ds(g * 16, 16)] = v >> 2
                off_v[pl.ds(slot * TB + g * 16, 16)] = (v & 3) << 5
            pltpu.async_copy(
                w4_hbm.at[idx4_v.at[slot]],
                gbuf.at[pl.ds(slot * TB, TB)],
                gsem[slot],
            )

        def drain_gather(slot):
            pltpu.make_async_copy(
                w4_hbm.at[pl.ds(0, TB)],
                gbuf.at[pl.ds(slot * TB, TB)],
                gsem[slot],
            ).wait()

        def transpose(slot):
            gv = gbuf.at[pl.ds(slot * TB, TB)]
            vjs = [g * 16 + iota for g in range(TB // 16)]
            vcols = [off_v[pl.ds(slot * TB + g * 16, 16)]
                     for g in range(TB // 16)]

            def dbody(i, carry):
                d0 = i * 2
                vs = [
                    plsc.load_gather(gv, [vjs[g], vcols[g] + d0 + dd])
                    for dd in range(2)
                    for g in range(TB // 16)
                ]
                k = 0
                for dd in range(2):
                    for g in range(TB // 16):
                        tbuf[slot * D + d0 + dd, pl.ds(g * 16, 16)] = vs[k]
                        k += 1
                return carry

            lax.fori_loop(0, D // 2, dbody, 0)

        def writeback(c, slot):
            s_ = c // NH
            boff = (c % NH) * TB
            pltpu.async_copy(
                tbuf.at[pl.ds(slot * D, D)],
                out_hbm.at[s_, :, pl.ds(b0w + boff, TB)],
                wsem[slot],
            )

        def drain_wb(slot):
            pltpu.make_async_copy(
                tbuf.at[pl.ds(slot * D, D)],
                out_hbm.at[0, :, pl.ds(0, TB)],
                wsem[slot],
            ).wait()

        for p in range(NS - 1):
            prep_fire(p, p)

        def outer(c, carry):
            for slot in range(NS):
                cc = c + slot
                nxt = cc + NS - 1

                @pl.when(nxt < NT)
                def _():
                    prep_fire(nxt, (slot - 1) % NS)

                drain_gather(slot)

                @pl.when(cc >= NS)
                def _():
                    drain_wb(slot)

                transpose(slot)
                writeback(cc, slot)
            return carry

        lax.fori_loop(0, NT // NS, lambda i, cr: outer(i * NS, cr), 0)
        for p in range(NS):
            drain_wb(p)

    return body


def kernel(token_ids, weights):
    B0, S = token_ids.shape
    V, D = weights.shape
    idxp = jnp.pad(token_ids.T.astype(jnp.int32), ((0, (-S) % 8), (0, 0)))
    w4 = weights.reshape(V * D // 128, 128)
    oT = _build(S, D, B0)(idxp, w4)
    return oT.transpose(2, 0, 1)


# speedup vs baseline: 1.2098x; 1.0078x over previous
import functools

import jax
import jax.numpy as jnp
from jax import lax
from jax.experimental import pallas as pl
from jax.experimental.pallas import tpu as pltpu
from jax.experimental.pallas import tpu_sc as plsc

NW = 32
SB = 512
TB = 128
NH = SB // TB
NS = 4


def _build(S, D, Bc):
    Sp = S + (-S) % 8
    NT = S * NH
    mesh = plsc.VectorSubcoreMesh(core_axis_name="c", subcore_axis_name="s")

    @functools.partial(
        pl.kernel,
        mesh=mesh,
        out_type=jax.ShapeDtypeStruct((S, D, Bc), jnp.float32),
        compiler_params=pltpu.CompilerParams(
            use_tc_tiling_on_sc=True, needs_layout_passes=False
        ),
        scratch_types=[
            pltpu.VMEM((Sp, SB), jnp.int32),
            pltpu.VMEM((NS, TB), jnp.int32),
            pltpu.VMEM((NS * TB,), jnp.int32),
            pltpu.VMEM((NS * TB, 128), jnp.float32),
            pltpu.VMEM((NS * D, TB), jnp.float32),
        ] + [pltpu.SemaphoreType.DMA] * (2 * NS),
    )
    def body(idx_hbm, w4_hbm, out_hbm, idx_v, idx4_v, off_v, gbuf, tbuf,
             *sems):
        gsem = sems[:NS]
        wsem = sems[NS:]
        wid = lax.axis_index("s") * 2 + lax.axis_index("c")
        b0w = wid * SB

        pltpu.sync_copy(idx_hbm.at[:, pl.ds(b0w, SB)], idx_v)

        iota = lax.iota(jnp.int32, 16)

        def prep_fire(c, slot):
            s_ = c // NH
            boff = (c % NH) * TB
            for g in range(TB // 16):
                v = idx_v[s_, pl.ds(boff + g * 16, 16)]
                idx4_v[slot, pl.ds(g * 16, 16)] = v >> 2
                off_v[pl.ds(slot * TB + g * 16, 16)] = (v & 3) << 5
            pltpu.async_copy(
                w4_hbm.at[idx4_v.at[slot]],
                gbuf.at[pl.ds(slot * TB, TB)],
                gsem[slot],
            )

        def drain_gather(slot):
            pltpu.make_async_copy(
                w4_hbm.at[pl.ds(0, TB)],
                gbuf.at[pl.ds(slot * TB, TB)],
                gsem[slot],
            ).wait()

        def transpose(slot):
            gv = gbuf.at[pl.ds(slot * TB, TB)]
            vjs = [g * 16 + iota for g in range(TB // 16)]
            vcols = [off_v[pl.ds(slot * TB + g * 16, 16)]
                     for g in range(TB // 16)]

            def dbody(i, carry):
                d0 = i * 4
                vs = [
                    plsc.load_gather(gv, [vjs[g], vcols[g] + d0 + dd])
                    for dd in range(4)
                    for g in range(TB // 16)
                ]
                k = 0
                for dd in range(4):
                    for g in range(TB // 16):
                        tbuf[slot * D + d0 + dd, pl.ds(g * 16, 16)] = vs[k]
                        k += 1
                return carry

            lax.fori_loop(0, D // 4, dbody, 0)

        def writeback(c, slot):
            s_ = c // NH
            boff = (c % NH) * TB
            pltpu.async_copy(
                tbuf.at[pl.ds(slot * D, D)],
                out_hbm.at[s_, :, pl.ds(b0w + boff, TB)],
                wsem[slot],
            )

        def drain_wb(slot):
            pltpu.make_async_copy(
                tbuf.at[pl.ds(slot * D, D)],
                out_hbm.at[0, :, pl.ds(0, TB)],
                wsem[slot],
            ).wait()

        for p in range(NS - 1):
            prep_fire(p, p)

        def outer(c, carry):
            for slot in range(NS):
                cc = c + slot
                nxt = cc + NS - 1

                @pl.when(nxt < NT)
                def _():
                    prep_fire(nxt, (slot - 1) % NS)

                drain_gather(slot)

                @pl.when(cc >= NS)
                def _():
                    drain_wb(slot)

                transpose(slot)
                writeback(cc, slot)
            return carry

        lax.fori_loop(0, NT // NS, lambda i, cr: outer(i * NS, cr), 0)
        for p in range(NS):
            drain_wb(p)

    return body


def kernel(token_ids, weights):
    B0, S = token_ids.shape
    V, D = weights.shape
    idxp = jnp.pad(token_ids.T.astype(jnp.int32), ((0, (-S) % 8), (0, 0)))
    w4 = weights.reshape(V * D // 128, 128)
    oT = _build(S, D, B0)(idxp, w4)
    return oT.transpose(2, 0, 1)
